# baseline (speedup 1.0000x reference)
import functools

import jax
import jax.numpy as jnp
from jax.experimental import pallas as pl
from jax.experimental.pallas import tpu as pltpu


def _pool_proj_bitcast_kernel(y_ref, w_ref, o_ref, *, inv_total, h, m):
    y = y_ref[...].astype(jnp.float32)
    hm = h // m
    parts = [jnp.sum(y[:, q * hm:(q + 1) * hm], axis=(1, 2, 3)) for q in range(m)]
    t = jnp.concatenate(parts, axis=-1)
    z = jax.lax.dot_general(t, w_ref[...].astype(jnp.float32),
                            dimension_numbers=(((1,), (1,)), ((), ())),
                            preferred_element_type=jnp.float32)
    o_ref[:, 0, :] = (z * inv_total).astype(o_ref.dtype)


def _pool_proj_collapse_kernel(y_ref, w_ref, o_ref, *, inv_total, hw, m):
    y = y_ref[...].astype(jnp.float32)
    parts = [jnp.sum(y[:, :, q * hw:(q + 1) * hw], axis=-1) for q in range(m)]
    t = jnp.concatenate(parts, axis=-1)
    z = jax.lax.dot_general(t, w_ref[...].astype(jnp.float32),
                            dimension_numbers=(((1,), (1,)), ((), ())),
                            preferred_element_type=jnp.float32)
    o_ref[...] = (z * inv_total).astype(o_ref.dtype)


def _pool_proj_tiled_kernel(x_ref, w_ref, o_ref, acc_ref, *,
                            inv_total, hw, thw, need_mask):
    hj = pl.program_id(1)

    @pl.when(hj == 0)
    def _():
        acc_ref[...] = jnp.zeros_like(acc_ref)

    x = x_ref[...].astype(jnp.float32)
    if need_mask:
        lane = jax.lax.broadcasted_iota(jnp.int32, x.shape, dimension=3)
        x = jnp.where(lane < (hw - hj * thw), x, 0.0)
    acc_ref[...] += jnp.sum(jnp.sum(x, axis=1), axis=-1)

    @pl.when(hj == pl.num_programs(1) - 1)
    def _():
        w = w_ref[...].astype(jnp.float32)
        z = jax.lax.dot_general(acc_ref[...] * inv_total, w,
                                dimension_numbers=(((1,), (1,)), ((), ())),
                                preferred_element_type=jnp.float32)
        o_ref[0, :, :] = z.astype(o_ref.dtype)


def _largest_aligned_divisor(total, align, limit):
    limit = min(limit, total)
    d = (limit // align) * align
    while d >= align:
        if total % d == 0:
            return d
        d -= align
    return None


def _pick_tb(b, row_bytes, budget):
    tb = None
    for d in range(1, b + 1):
        if b % d == 0 and d * row_bytes <= budget and (d % 8 == 0 or d == b):
            tb = d
    return tb


def _permuted_weight(proj_weight, c, m):
    co = jnp.arange(c)
    return jnp.concatenate(
        [proj_weight[:, (co * m + q) % c] for q in range(m)], axis=1)


def kernel(x, proj_weight):
    b, c, h, w, m = x.shape
    out_c = proj_weight.shape[0]
    hw = h * w
    itemsize = jnp.dtype(x.dtype).itemsize
    inv_total = 1.0 / float(m * hw)

    row_bytes = m * c * hw * itemsize
    block_target = 8 * 1024 * 1024
    vmem_cap = 64 * 1024 * 1024
    tb = None
    for d in range(1, b + 1):
        if b % d == 0 and d * row_bytes <= block_target and (b // d) >= 2:
            tb = d

    w_bytes = out_c * m * c * jnp.dtype(proj_weight.dtype).itemsize

    if tb is not None and h % m == 0:
        n_b = b // tb
        xt = jnp.transpose(x, (0, 2, 3, 4, 1))
        wcat = _permuted_weight(proj_weight, c, m)

        block_bytes = tb * row_bytes
        vmem_need = 2 * block_bytes + 2 * w_bytes + 2 * tb * out_c * itemsize + (2 << 20)
        vmem_limit = min(max(vmem_need, 8 << 20), int(vmem_cap * 0.9))

        z = pl.pallas_call(
            functools.partial(_pool_proj_bitcast_kernel,
                              inv_total=inv_total, h=h, m=m),
            out_shape=jax.ShapeDtypeStruct((b, 1, out_c), x.dtype),
            grid_spec=pltpu.PrefetchScalarGridSpec(
                num_scalar_prefetch=0,
                grid=(n_b,),
                in_specs=[pl.BlockSpec((tb, h, w, m, c),
                                       lambda bi: (bi, 0, 0, 0, 0)),
                          pl.BlockSpec((out_c, m * c), lambda bi: (0, 0))],
                out_specs=pl.BlockSpec((tb, 1, out_c), lambda bi: (bi, 0, 0))),
            compiler_params=pltpu.CompilerParams(
                dimension_semantics=("parallel",),
                vmem_limit_bytes=vmem_limit),
        )(xt, wcat)
        return jnp.swapaxes(z, 1, 2)

    if tb is not None:
        n_b = b // tb
        y = x.reshape(b, c, hw * m)
        wcat = _permuted_weight(proj_weight, c, m)

        block_bytes = tb * row_bytes
        vmem_need = 2 * block_bytes + 2 * w_bytes + 2 * tb * out_c * itemsize + (2 << 20)
        vmem_limit = min(max(vmem_need, 8 << 20), int(vmem_cap * 0.9))

        z = pl.pallas_call(
            functools.partial(_pool_proj_collapse_kernel,
                              inv_total=inv_total, hw=hw, m=m),
            out_shape=jax.ShapeDtypeStruct((b, out_c), x.dtype),
            grid_spec=pltpu.PrefetchScalarGridSpec(
                num_scalar_prefetch=0,
                grid=(n_b,),
                in_specs=[pl.BlockSpec((tb, c, hw * m), lambda bi: (bi, 0, 0)),
                          pl.BlockSpec((out_c, m * c), lambda bi: (0, 0))],
                out_specs=pl.BlockSpec((tb, out_c), lambda bi: (bi, 0))),
            compiler_params=pltpu.CompilerParams(
                dimension_semantics=("parallel",),
                vmem_limit_bytes=vmem_limit),
        )(y, wcat)
        return z[..., None]

    x4 = x.reshape(b, m, c, hw)
    cap_hw = max(block_target // max(m * c * itemsize, 1), 128)
    if hw <= cap_hw:
        thw = hw
    else:
        cap128 = max((cap_hw // 128) * 128, 128)
        thw = _largest_aligned_divisor(hw, 128, cap128) or cap128
    n_hw = -(-hw // thw)
    need_mask = (hw % thw) != 0

    block_bytes = m * c * thw * itemsize
    wp_bytes = out_c * c * jnp.dtype(proj_weight.dtype).itemsize
    vmem_need = (2 * block_bytes + 2 * wp_bytes + 2 * out_c * itemsize
                 + c * 4 + (2 << 20))
    vmem_limit = min(max(vmem_need, 8 << 20), int(vmem_cap * 0.9))

    z = pl.pallas_call(
        functools.partial(_pool_proj_tiled_kernel, inv_total=inv_total,
                          hw=hw, thw=thw, need_mask=need_mask),
        out_shape=jax.ShapeDtypeStruct((b, 1, out_c), x.dtype),
        grid_spec=pltpu.PrefetchScalarGridSpec(
            num_scalar_prefetch=0,
            grid=(b, n_hw),
            in_specs=[pl.BlockSpec((1, m, c, thw), lambda bi, hj: (bi, 0, 0, hj)),
                      pl.BlockSpec((out_c, c), lambda bi, hj: (0, 0))],
            out_specs=pl.BlockSpec((1, 1, out_c), lambda bi, hj: (bi, 0, 0)),
            scratch_shapes=[pltpu.VMEM((1, c), jnp.float32)]),
        compiler_params=pltpu.CompilerParams(
            dimension_semantics=("parallel", "arbitrary"),
            vmem_limit_bytes=vmem_limit),
    )(x4, proj_weight)

    return jnp.swapaxes(z, 1, 2)


# speedup vs baseline: 1.0463x; 1.0463x over previous
import functools

import jax
import jax.numpy as jnp
from jax.experimental import pallas as pl
from jax.experimental.pallas import tpu as pltpu


def _pool_proj_bitcast_kernel(y_ref, w_ref, o_ref, *, inv_total, h, m):
    y = y_ref[...].astype(jnp.float32)
    hm = h // m
    parts = [jnp.sum(y[:, q * hm:(q + 1) * hm], axis=(1, 2, 3)) for q in range(m)]
    t = jnp.concatenate(parts, axis=-1)
    z = jax.lax.dot_general(t, w_ref[...].astype(jnp.float32),
                            dimension_numbers=(((1,), (1,)), ((), ())),
                            preferred_element_type=jnp.float32)
    o_ref[:, 0, :] = (z * inv_total).astype(o_ref.dtype)


def _pool_proj_collapse_kernel(y_ref, w_ref, o_ref, *, inv_total, hw, m):
    y = y_ref[...].astype(jnp.float32)
    parts = [jnp.sum(y[:, :, q * hw:(q + 1) * hw], axis=-1) for q in range(m)]
    t = jnp.concatenate(parts, axis=-1)
    z = jax.lax.dot_general(t, w_ref[...].astype(jnp.float32),
                            dimension_numbers=(((1,), (1,)), ((), ())),
                            preferred_element_type=jnp.float32)
    o_ref[...] = (z * inv_total).astype(o_ref.dtype)


def _pool_proj_tiled_kernel(x_ref, w_ref, o_ref, acc_ref, *,
                            inv_total, hw, thw, need_mask):
    hj = pl.program_id(1)

    @pl.when(hj == 0)
    def _():
        acc_ref[...] = jnp.zeros_like(acc_ref)

    x = x_ref[...].astype(jnp.float32)
    if need_mask:
        lane = jax.lax.broadcasted_iota(jnp.int32, x.shape, dimension=3)
        x = jnp.where(lane < (hw - hj * thw), x, 0.0)
    acc_ref[...] += jnp.sum(jnp.sum(x, axis=1), axis=-1)

    @pl.when(hj == pl.num_programs(1) - 1)
    def _():
        w = w_ref[...].astype(jnp.float32)
        z = jax.lax.dot_general(acc_ref[...] * inv_total, w,
                                dimension_numbers=(((1,), (1,)), ((), ())),
                                preferred_element_type=jnp.float32)
        o_ref[0, :, :] = z.astype(o_ref.dtype)


def _largest_aligned_divisor(total, align, limit):
    limit = min(limit, total)
    d = (limit // align) * align
    while d >= align:
        if total % d == 0:
            return d
        d -= align
    return None


def _pick_tb(b, row_bytes, budget):
    tb = None
    for d in range(1, b + 1):
        if b % d == 0 and d * row_bytes <= budget and (d % 8 == 0 or d == b):
            tb = d
    return tb


def _permuted_weight(proj_weight, c, m):
    co = jnp.arange(c)
    return jnp.concatenate(
        [proj_weight[:, (co * m + q) % c] for q in range(m)], axis=1)


def kernel(x, proj_weight):
    b, c, h, w, m = x.shape
    out_c = proj_weight.shape[0]
    hw = h * w
    itemsize = jnp.dtype(x.dtype).itemsize
    inv_total = 1.0 / float(m * hw)

    row_bytes = m * c * hw * itemsize
    block_target = 16 * 1024 * 1024
    vmem_cap = 64 * 1024 * 1024
    tb = None
    for d in range(1, b + 1):
        if b % d == 0 and d * row_bytes <= block_target and (b // d) >= 2:
            tb = d

    w_bytes = out_c * m * c * jnp.dtype(proj_weight.dtype).itemsize

    if tb is not None and h % m == 0:
        n_b = b // tb
        xt = jnp.transpose(x, (0, 2, 3, 4, 1))
        wcat = _permuted_weight(proj_weight, c, m)

        block_bytes = tb * row_bytes
        vmem_need = 2 * block_bytes + 2 * w_bytes + 2 * tb * out_c * itemsize + (2 << 20)
        vmem_limit = min(max(vmem_need, 8 << 20), int(vmem_cap * 0.9))

        z = pl.pallas_call(
            functools.partial(_pool_proj_bitcast_kernel,
                              inv_total=inv_total, h=h, m=m),
            out_shape=jax.ShapeDtypeStruct((b, 1, out_c), x.dtype),
            grid_spec=pltpu.PrefetchScalarGridSpec(
                num_scalar_prefetch=0,
                grid=(n_b,),
                in_specs=[pl.BlockSpec((tb, h, w, m, c),
                                       lambda bi: (bi, 0, 0, 0, 0)),
                          pl.BlockSpec((out_c, m * c), lambda bi: (0, 0))],
                out_specs=pl.BlockSpec((tb, 1, out_c), lambda bi: (bi, 0, 0))),
            compiler_params=pltpu.CompilerParams(
                dimension_semantics=("parallel",),
                vmem_limit_bytes=vmem_limit),
        )(xt, wcat)
        return jnp.swapaxes(z, 1, 2)

    if tb is not None:
        n_b = b // tb
        y = x.reshape(b, c, hw * m)
        wcat = _permuted_weight(proj_weight, c, m)

        block_bytes = tb * row_bytes
        vmem_need = 2 * block_bytes + 2 * w_bytes + 2 * tb * out_c * itemsize + (2 << 20)
        vmem_limit = min(max(vmem_need, 8 << 20), int(vmem_cap * 0.9))

        z = pl.pallas_call(
            functools.partial(_pool_proj_collapse_kernel,
                              inv_total=inv_total, hw=hw, m=m),
            out_shape=jax.ShapeDtypeStruct((b, out_c), x.dtype),
            grid_spec=pltpu.PrefetchScalarGridSpec(
                num_scalar_prefetch=0,
                grid=(n_b,),
                in_specs=[pl.BlockSpec((tb, c, hw * m), lambda bi: (bi, 0, 0)),
                          pl.BlockSpec((out_c, m * c), lambda bi: (0, 0))],
                out_specs=pl.BlockSpec((tb, out_c), lambda bi: (bi, 0))),
            compiler_params=pltpu.CompilerParams(
                dimension_semantics=("parallel",),
                vmem_limit_bytes=vmem_limit),
        )(y, wcat)
        return z[..., None]

    x4 = x.reshape(b, m, c, hw)
    cap_hw = max(block_target // max(m * c * itemsize, 1), 128)
    if hw <= cap_hw:
        thw = hw
    else:
        cap128 = max((cap_hw // 128) * 128, 128)
        thw = _largest_aligned_divisor(hw, 128, cap128) or cap128
    n_hw = -(-hw // thw)
    need_mask = (hw % thw) != 0

    block_bytes = m * c * thw * itemsize
    wp_bytes = out_c * c * jnp.dtype(proj_weight.dtype).itemsize
    vmem_need = (2 * block_bytes + 2 * wp_bytes + 2 * out_c * itemsize
                 + c * 4 + (2 << 20))
    vmem_limit = min(max(vmem_need, 8 << 20), int(vmem_cap * 0.9))

    z = pl.pallas_call(
        functools.partial(_pool_proj_tiled_kernel, inv_total=inv_total,
                          hw=hw, thw=thw, need_mask=need_mask),
        out_shape=jax.ShapeDtypeStruct((b, 1, out_c), x.dtype),
        grid_spec=pltpu.PrefetchScalarGridSpec(
            num_scalar_prefetch=0,
            grid=(b, n_hw),
            in_specs=[pl.BlockSpec((1, m, c, thw), lambda bi, hj: (bi, 0, 0, hj)),
                      pl.BlockSpec((out_c, c), lambda bi, hj: (0, 0))],
            out_specs=pl.BlockSpec((1, 1, out_c), lambda bi, hj: (bi, 0, 0)),
            scratch_shapes=[pltpu.VMEM((1, c), jnp.float32)]),
        compiler_params=pltpu.CompilerParams(
            dimension_semantics=("parallel", "arbitrary"),
            vmem_limit_bytes=vmem_limit),
    )(x4, proj_weight)

    return jnp.swapaxes(z, 1, 2)
